# baseline (speedup 1.0000x reference)
import functools

import jax
import jax.numpy as jnp
from jax import lax
from jax.experimental import pallas as pl
from jax.experimental.pallas import tpu as pltpu
from jax.experimental.pallas import tpu_sc as plsc

BATCH = 4096
SEQ = 50
D = 64
PAIRS = 500000
NC, NS = 2, 16
NW = NC * NS
B_PER_W = BATCH // NW
CB = 4
N_CHUNKS = B_PER_W // CB


def _lane_shuffle(v, perm):
    return v.at[perm].get(mode="promise_in_bounds")


def _normalize_chunk(gbuf, hb_v, cbuf):
    lanes = lax.iota(jnp.int32, 16)
    perms = [lanes ^ k for k in (1, 2, 4, 8)]

    @plsc.parallel_loop(0, CB * SEQ, 1, unroll=2)
    def body(i):
        bi = i // SEQ
        si = i % SEQ
        sw = (si // 16) * 16
        hwin = hb_v[bi, pl.ds(sw, 16)]
        hsp = _lane_shuffle(hwin, jnp.broadcast_to(si - sw, (16,)))
        hf = hsp.astype(jnp.float32)
        a0 = gbuf[bi, si, pl.ds(0, 16)]
        a1 = gbuf[bi, si, pl.ds(16, 16)]
        a2 = gbuf[bi, si, pl.ds(32, 16)]
        a3 = gbuf[bi, si, pl.ds(48, 16)]
        b0 = gbuf[bi, si, pl.ds(64, 16)]
        b1 = gbuf[bi, si, pl.ds(80, 16)]
        b2 = gbuf[bi, si, pl.ds(96, 16)]
        b3 = gbuf[bi, si, pl.ds(112, 16)]
        v0 = a0 + (b0 - a0) * hf
        v1 = a1 + (b1 - a1) * hf
        v2 = a2 + (b2 - a2) * hf
        v3 = a3 + (b3 - a3) * hf
        s = v0 * v0 + v1 * v1 + v2 * v2 + v3 * v3
        for p in perms:
            s = s + _lane_shuffle(s, p)
        ib = lax.bitcast_convert_type(s, jnp.int32)
        ib = jnp.int32(0x5F3759DF) - (ib >> 1)
        r = lax.bitcast_convert_type(ib, jnp.float32)
        r = r * (1.5 - 0.5 * s * r * r)
        inv = r * (1.5 - 0.5 * s * r * r)
        cbuf[bi, si, pl.ds(0, 16)] = v0 * inv
        cbuf[bi, si, pl.ds(16, 16)] = v1 * inv
        cbuf[bi, si, pl.ds(32, 16)] = v2 * inv
        cbuf[bi, si, pl.ds(48, 16)] = v3 * inv


def _sc_gather_norm(px, hb, table2):
    mesh = plsc.VectorSubcoreMesh(core_axis_name="c", subcore_axis_name="s")

    @functools.partial(
        pl.kernel,
        mesh=mesh,
        out_type=jax.ShapeDtypeStruct((BATCH, SEQ, D), jnp.float32),
        compiler_params=pltpu.CompilerParams(use_tc_tiling_on_sc=False),
        scratch_types=[
            pltpu.VMEM((CB, SEQ), jnp.int32),
            pltpu.VMEM((CB, SEQ), jnp.int32),
            pltpu.VMEM((CB, 64), jnp.int32),
            pltpu.VMEM((CB, 64), jnp.int32),
            pltpu.VMEM((CB, SEQ, 2 * D), jnp.float32),
            pltpu.VMEM((CB, SEQ, 2 * D), jnp.float32),
            pltpu.VMEM((CB, SEQ, D), jnp.float32),
            pltpu.SemaphoreType.DMA,
            pltpu.SemaphoreType.DMA,
            pltpu.SemaphoreType.DMA,
            pltpu.SemaphoreType.DMA,
        ],
    )
    def k(px_hbm, hb_hbm, tab_hbm, out_hbm,
          px0, px1, hb0, hb1, gb0, gb1, cbuf, g0, g1, w0, w1):
        pxb = (px0, px1)
        hbb = (hb0, hb1)
        gbuf = (gb0, gb1)
        gsem = (g0, g1)
        wsem = (w0, w1)
        wid = lax.axis_index("s") * NC + lax.axis_index("c")
        batch0 = wid * B_PER_W

        def stage_idx(c, b):
            pltpu.sync_copy(px_hbm.at[pl.ds(batch0 + c * CB, CB)], pxb[b])
            pltpu.sync_copy(hb_hbm.at[pl.ds(batch0 + c * CB, CB)], hbb[b])

        def start_gathers(c, b):
            return [
                pltpu.async_copy(
                    tab_hbm.at[pxb[b].at[j]],
                    gbuf[b].at[j],
                    gsem[b],
                )
                for j in range(CB)
            ]

        stage_idx(0, 0)
        gcps = {0: start_gathers(0, 0)}
        wcp = None
        for c in range(N_CHUNKS):
            b, nb = c % 2, (c + 1) % 2
            if c + 1 < N_CHUNKS:
                stage_idx(c + 1, nb)
                gcps[c + 1] = start_gathers(c + 1, nb)
            for cp in gcps.pop(c):
                cp.wait()
            if wcp is not None:
                wcp.wait()
            _normalize_chunk(gbuf[b], hbb[b], cbuf)
            wcp = pltpu.async_copy(
                cbuf,
                out_hbm.at[pl.ds(batch0 + c * CB, CB)],
                wsem[b],
            )
        wcp.wait()

    return k(px, hb, table2)


def kernel(x, lengths, table):
    xi = x.astype(jnp.int32)
    px = xi >> 1
    hb = jnp.pad(xi & 1, ((0, 0), (0, 64 - SEQ)))
    cap_emb = _sc_gather_norm(px, hb, table.reshape(PAIRS, 2 * D))
    cap_len = jnp.asarray(lengths, dtype=jnp.int32)
    return (cap_emb, cap_len)
